# baseline (speedup 1.0000x reference)
import jax
import jax.numpy as jnp
from jax.experimental import pallas as pl

_W = 4


def _copy_body(in_ref, out_ref):
    out_ref[...] = in_ref[...]


def kernel(x):
    B, S, D = x.shape
    So = S // _W
    x2 = x.reshape(B, So, _W * D)
    blk = 256
    out = pl.pallas_call(
        _copy_body,
        grid=(B, So // blk),
        in_specs=[pl.BlockSpec((1, blk, D), lambda b, i: (b, i, 0))],
        out_specs=pl.BlockSpec((1, blk, D), lambda b, i: (b, i, 0)),
        out_shape=jax.ShapeDtypeStruct((B, So, D), x.dtype),
    )(x2)
    return out
